# baseline (speedup 1.0000x reference)
import functools

import jax
import jax.numpy as jnp
from jax import lax
from jax.experimental import pallas as pl
from jax.experimental.pallas import tpu as pltpu
from jax.experimental.pallas import tpu_sc as plsc

_B_ROWS = 16384 * 26
_DIM = 64
_NC, _NS = 2, 16
_NW = _NC * _NS
_BPW = _B_ROWS // _NW
_CH = 128
_NCH = _BPW // _CH
_NBUF = 2

_mesh = plsc.VectorSubcoreMesh(core_axis_name="c", subcore_axis_name="s")


@functools.partial(
    pl.kernel,
    out_type=jax.ShapeDtypeStruct((_B_ROWS, _DIM), jnp.float32),
    mesh=_mesh,
    scratch_types=[
        pltpu.VMEM((_NCH, _CH), jnp.int32),
        pltpu.VMEM((_NBUF, _CH, _DIM), jnp.float32),
        pltpu.SemaphoreType.DMA,
        pltpu.SemaphoreType.DMA,
    ],
    compiler_params=pltpu.CompilerParams(use_tc_tiling_on_sc=False),
)
def _sc_gather(idx_hbm, table_hbm, out_hbm, idx_v, rows_v, sem0, sem1):
    wid = lax.axis_index("s") * _NC + lax.axis_index("c")
    base = wid * _BPW
    sems = (sem0, sem1)

    pltpu.sync_copy(idx_hbm.at[wid], idx_v)

    for b in range(_NBUF):
        pltpu.async_copy(table_hbm.at[idx_v.at[b]], rows_v.at[b], sems[b])

    def _wait_and_flush(g, b):
        pltpu.make_async_copy(
            table_hbm.at[idx_v.at[0]], rows_v.at[b], sems[b]
        ).wait()
        pltpu.sync_copy(rows_v.at[b], out_hbm.at[pl.ds(base + g * _CH, _CH)])

    @pl.loop(0, _NCH - _NBUF, step=_NBUF)
    def _body(jj):
        for b in range(_NBUF):
            g = jj + b
            _wait_and_flush(g, b)
            pltpu.async_copy(
                table_hbm.at[idx_v.at[g + _NBUF]], rows_v.at[b], sems[b]
            )

    for b in range(_NBUF):
        _wait_and_flush(_NCH - _NBUF + b, b)


def kernel(x, weight):
    idx = x.astype(jnp.int32).reshape(_NW, _NCH, _CH)
    out = _sc_gather(idx, weight)
    return out.reshape(x.shape[0], x.shape[1], _DIM)


# speedup vs baseline: 1.0095x; 1.0095x over previous
import functools

import jax
import jax.numpy as jnp
from jax import lax
from jax.experimental import pallas as pl
from jax.experimental.pallas import tpu as pltpu
from jax.experimental.pallas import tpu_sc as plsc

_B_ROWS = 16384 * 26
_DIM = 64
_NC, _NS = 2, 16
_NW = _NC * _NS
_BPW = _B_ROWS // _NW
_CH = 256
_NCH = _BPW // _CH
_NBUF = 4

_mesh = plsc.VectorSubcoreMesh(core_axis_name="c", subcore_axis_name="s")


@functools.partial(
    pl.kernel,
    out_type=jax.ShapeDtypeStruct((_B_ROWS, _DIM), jnp.float32),
    mesh=_mesh,
    scratch_types=[
        pltpu.VMEM((_NCH, _CH), jnp.int32),
        pltpu.VMEM((_NBUF, _CH, _DIM), jnp.float32),
        [pltpu.SemaphoreType.DMA] * _NBUF,
        [pltpu.SemaphoreType.DMA] * _NBUF,
    ],
    compiler_params=pltpu.CompilerParams(use_tc_tiling_on_sc=False),
)
def _sc_gather(idx_hbm, table_hbm, out_hbm, idx_v, rows_v, gsems, wsems):
    wid = lax.axis_index("s") * _NC + lax.axis_index("c")
    base = wid * _BPW

    pltpu.sync_copy(idx_hbm.at[wid], idx_v)

    def _start_gather(g, b):
        pltpu.async_copy(table_hbm.at[idx_v.at[g]], rows_v.at[b], gsems[b])

    def _wait_gather(b):
        pltpu.make_async_copy(
            table_hbm.at[idx_v.at[0]], rows_v.at[b], gsems[b]
        ).wait()

    def _out_slot(g):
        return out_hbm.at[pl.ds(base + g * _CH, _CH)]

    def _wait_write(g, b):
        pltpu.make_async_copy(rows_v.at[b], _out_slot(g), wsems[b]).wait()

    for b in range(_NBUF):
        _start_gather(b, b)

    @pl.loop(0, _NCH - _NBUF, step=_NBUF)
    def _body(jj):
        for b in range(_NBUF):
            _wait_gather(b)
            pltpu.async_copy(rows_v.at[b], _out_slot(jj + b), wsems[b])
        for b in range(_NBUF):
            _wait_write(jj + b, b)
            _start_gather(jj + b + _NBUF, b)

    for b in range(_NBUF):
        g = _NCH - _NBUF + b
        _wait_gather(b)
        pltpu.sync_copy(rows_v.at[b], _out_slot(g))


def kernel(x, weight):
    idx = x.astype(jnp.int32).reshape(_NW, _NCH, _CH)
    out = _sc_gather(idx, weight)
    return out.reshape(x.shape[0], x.shape[1], _DIM)


# speedup vs baseline: 1.0097x; 1.0002x over previous
import functools

import jax
import jax.numpy as jnp
from jax import lax
from jax.experimental import pallas as pl
from jax.experimental.pallas import tpu as pltpu
from jax.experimental.pallas import tpu_sc as plsc

_NI_TOT = 16384
_NJ = 26
_B_ROWS = _NI_TOT * _NJ
_DIM = 64
_NC, _NS = 2, 16
_NW = _NC * _NS
_BPW = _B_ROWS // _NW
_NI = 8
_CH = _NI * _NJ
_NCH = _BPW // _CH
_IPW = _NI_TOT // _NW
_NBUF = 4

_mesh = plsc.VectorSubcoreMesh(core_axis_name="c", subcore_axis_name="s")


@functools.partial(
    pl.kernel,
    out_type=jax.ShapeDtypeStruct((_NI_TOT, _NJ, _DIM), jnp.float32),
    mesh=_mesh,
    scratch_types=[
        pltpu.VMEM((_NCH, _CH), jnp.int32),
        pltpu.VMEM((_NBUF, _CH, _DIM), jnp.float32),
        [pltpu.SemaphoreType.DMA] * _NBUF,
        [pltpu.SemaphoreType.DMA] * _NBUF,
    ],
    compiler_params=pltpu.CompilerParams(use_tc_tiling_on_sc=False),
)
def _sc_gather(idx_hbm, table_hbm, out_hbm, idx_v, rows_v, gsems, wsems):
    wid = lax.axis_index("s") * _NC + lax.axis_index("c")
    i_base = wid * _IPW

    pltpu.sync_copy(idx_hbm.at[wid], idx_v)

    def _start_gather(g, b):
        pltpu.async_copy(table_hbm.at[idx_v.at[g]], rows_v.at[b], gsems[b])

    def _wait_gather(b):
        pltpu.make_async_copy(
            table_hbm.at[idx_v.at[0]], rows_v.at[b], gsems[b]
        ).wait()

    def _start_writes(g, b):
        for di in range(_NI):
            pltpu.async_copy(
                rows_v.at[b, pl.ds(di * _NJ, _NJ)],
                out_hbm.at[i_base + g * _NI + di],
                wsems[b],
            )

    def _wait_writes(g, b):
        for di in range(_NI):
            pltpu.make_async_copy(
                rows_v.at[b, pl.ds(di * _NJ, _NJ)],
                out_hbm.at[i_base + g * _NI + di],
                wsems[b],
            ).wait()

    for b in range(_NBUF):
        _start_gather(b, b)

    @pl.loop(0, _NCH - _NBUF, step=_NBUF)
    def _body(jj):
        for b in range(_NBUF):
            _wait_gather(b)
            _start_writes(jj + b, b)
        for b in range(_NBUF):
            _wait_writes(jj + b, b)
            _start_gather(jj + b + _NBUF, b)

    for b in range(_NBUF):
        g = _NCH - _NBUF + b
        _wait_gather(b)
        _start_writes(g, b)
        _wait_writes(g, b)


def kernel(x, weight):
    idx = x.astype(jnp.int32).reshape(_NW, _NCH, _CH)
    return _sc_gather(idx, weight)
